# baseline (speedup 1.0000x reference)
import functools

import jax
import jax.numpy as jnp
from jax import lax
from jax.experimental import pallas as pl
from jax.experimental.pallas import tpu as pltpu
from jax.experimental.pallas import tpu_sc as plsc

B = 16384
V = 1000000
D = 64
C = 20
NNEG = 50

NC = 2
NS = 16
NW = NC * NS
EPW = B // NW
NB = 8
NCH = EPW // NB
LANES = 16
ND = D // LANES
NG = 64


def _row(ref, r, dd):
    return ref[r, pl.ds(dd * LANES, LANES)]


def _dot_all_lanes(cv, ref, r, rots):
    p = cv[0] * _row(ref, r, 0)
    for dd in range(1, ND):
        p = p + cv[dd] * _row(ref, r, dd)
    for rot in rots:
        p = p + p.at[rot].get(mode="promise_in_bounds", unique_indices=True)
    return p


def _make_kernel():
    mesh = plsc.VectorSubcoreMesh(core_axis_name="c", subcore_axis_name="s")

    @functools.partial(
        pl.kernel,
        out_type=[
            jax.ShapeDtypeStruct((B,), jnp.float32),
            jax.ShapeDtypeStruct((B, NNEG), jnp.float32),
        ],
        mesh=mesh,
        compiler_params=pltpu.CompilerParams(use_tc_tiling_on_sc=False,
                                             needs_layout_passes=False),
        scratch_types=[
            pltpu.VMEM((NB, C), jnp.int32),
            pltpu.VMEM((NB, C), jnp.int32),
            pltpu.VMEM((NB, NNEG), jnp.int32),
            pltpu.VMEM((NB, NNEG), jnp.int32),
            pltpu.VMEM((NB, 1), jnp.int32),
            pltpu.VMEM((NB, 1), jnp.int32),
            pltpu.VMEM((NB * C, D), jnp.float32),
            pltpu.VMEM((NB * C, D), jnp.float32),
            pltpu.VMEM((NB * NNEG, D), jnp.float32),
            pltpu.VMEM((NB * NNEG, D), jnp.float32),
            pltpu.VMEM((NB, D), jnp.float32),
            pltpu.VMEM((NB, D), jnp.float32),
            pltpu.VMEM((NB, NNEG), jnp.float32),
            pltpu.VMEM((LANES,), jnp.float32),
            pltpu.SemaphoreType.DMA,
            pltpu.SemaphoreType.DMA,
        ],
    )
    def cbow(tidx_hbm, cidx_hbm, nidx_hbm, tw_hbm, cw_hbm, pos_hbm, neg_hbm,
             cidx0, cidx1, nidx0, nidx1, tidx0, tidx1,
             rows_c0, rows_c1, rows_n0, rows_n1, rows_tg0, rows_tg1,
             scores_buf, pos_buf, semA, semB):
        wid = lax.axis_index("s") * NC + lax.axis_index("c")
        wbase = wid * EPW

        bufs = ((cidx0, nidx0, tidx0, rows_c0, rows_n0, rows_tg0, semA),
                (cidx1, nidx1, tidx1, rows_c1, rows_n1, rows_tg1, semB))

        def gather_copies(buf):
            cidx, nidx, tidx, rows_c, rows_n, rows_tg, sem = buf
            copies = []
            for i in range(NB):
                copies.append(pltpu.make_async_copy(
                    cw_hbm.at[cidx.at[i]],
                    rows_c.at[pl.ds(i * C, C)], sem))
                copies.append(pltpu.make_async_copy(
                    tw_hbm.at[nidx.at[i]],
                    rows_n.at[pl.ds(i * NNEG, NNEG)], sem))
                copies.append(pltpu.make_async_copy(
                    tw_hbm.at[tidx.at[i]],
                    rows_tg.at[pl.ds(i, 1)], sem))
            return copies

        def issue(c, buf):
            @pl.when(c < NCH)
            def _():
                cidx, nidx, tidx, rows_c, rows_n, rows_tg, sem = buf
                base = wbase + c * NB
                pltpu.sync_copy(cidx_hbm.at[pl.ds(base, NB), :], cidx)
                pltpu.sync_copy(nidx_hbm.at[pl.ds(base, NB), :], nidx)
                pltpu.sync_copy(tidx_hbm.at[pl.ds(base, NB), :], tidx)
                for cp in gather_copies(buf):
                    cp.start()

        def drain(buf):
            for cp in gather_copies(buf):
                cp.wait()

        def compute(c, buf):
            _, _, _, rows_c, rows_n, rows_tg, _ = buf
            lane = lax.iota(jnp.int32, LANES)
            onehot = [lane == jj for jj in range(LANES)]
            rots = [(lane + sh) & (LANES - 1) for sh in (8, 4, 2, 1)]
            zero = jnp.zeros((LANES,), jnp.float32)

            def elem(i, pos_vec):
                ri = i * C
                acc = tuple(_row(rows_c, ri, dd) for dd in range(ND))
                for k in range(1, C):
                    acc = tuple(acc[dd] + _row(rows_c, ri + k, dd)
                                for dd in range(ND))
                scale = jnp.float32(1.0 / C)
                cv = tuple(a * scale for a in acc)

                ni = i * NNEG
                for g in range(NNEG // LANES):
                    sv = zero
                    for jj in range(LANES):
                        j = g * LANES + jj
                        s = _dot_all_lanes(cv, rows_n, ni + j, rots)
                        sv = jnp.where(onehot[jj], s, sv)
                    scores_buf[i, pl.ds(g * LANES, LANES)] = sv
                rem = NNEG % LANES
                rbase = (NNEG // LANES) * LANES
                sv = zero
                for jj in range(rem):
                    s = _dot_all_lanes(cv, rows_n, ni + rbase + jj, rots)
                    sv = jnp.where(onehot[jj], s, sv)
                plsc.store_scatter(
                    scores_buf,
                    [jnp.full((LANES,), i, jnp.int32), rbase + lane],
                    sv, mask=lane < rem)

                s = _dot_all_lanes(cv, rows_tg, i, rots)
                return jnp.where(lane == i, s, pos_vec)

            pos_vec = lax.fori_loop(0, NB, elem, zero)
            pos_buf[pl.ds(0, LANES)] = pos_vec
            gbase = wbase + c * NB
            pltpu.sync_copy(scores_buf, neg_hbm.at[pl.ds(gbase, NB), :])
            pltpu.sync_copy(pos_buf.at[pl.ds(0, NB)],
                            pos_hbm.at[pl.ds(gbase, NB)])

        issue(jnp.int32(0), bufs[0])
        issue(jnp.int32(1), bufs[1])

        def pair(k, _):
            c0 = 2 * k
            drain(bufs[0])
            compute(c0, bufs[0])
            issue(c0 + 2, bufs[0])
            c1 = c0 + 1
            drain(bufs[1])
            compute(c1, bufs[1])
            issue(c1 + 2, bufs[1])
            return 0

        lax.fori_loop(0, NCH // 2, pair, 0, unroll=False)

    return cbow


_cbow = _make_kernel()


def kernel(target_indices, context_indices, negative_indices, target_weight,
           context_weight):
    pos, neg = _cbow(target_indices, context_indices, negative_indices,
                     target_weight, context_weight)
    return pos.reshape(B, 1), neg


# speedup vs baseline: 1.0911x; 1.0911x over previous
import functools

import jax
import jax.numpy as jnp
from jax import lax
from jax.experimental import pallas as pl
from jax.experimental.pallas import tpu as pltpu
from jax.experimental.pallas import tpu_sc as plsc

B = 16384
V = 1000000
D = 64
C = 20
NNEG = 50
TN = 1 + NNEG

NC = 2
NS = 16
NW = NC * NS
EPW = B // NW
NB = 8
NCH = EPW // NB
LANES = 16
ND = D // LANES

CTX_SLICES = [(0, 80), (80, 80)]
TN_SLICES = [(0, 128), (128, 128), (256, 128), (384, 24)]


def _row(ref, r, dd):
    return ref[r, pl.ds(dd * LANES, LANES)]


def _dot_all_lanes(cv, ref, r, rots):
    p = cv[0] * _row(ref, r, 0)
    for dd in range(1, ND):
        p = p + cv[dd] * _row(ref, r, dd)
    for rot in rots:
        p = p + p.at[rot].get(mode="promise_in_bounds", unique_indices=True)
    return p


def _make_kernel():
    mesh = plsc.VectorSubcoreMesh(core_axis_name="c", subcore_axis_name="s")

    @functools.partial(
        pl.kernel,
        out_type=[
            jax.ShapeDtypeStruct((B,), jnp.float32),
            jax.ShapeDtypeStruct((B, NNEG), jnp.float32),
        ],
        mesh=mesh,
        compiler_params=pltpu.CompilerParams(use_tc_tiling_on_sc=False,
                                             needs_layout_passes=False),
        scratch_types=[
            pltpu.VMEM((EPW * C,), jnp.int32),
            pltpu.VMEM((EPW * TN,), jnp.int32),
            pltpu.VMEM((NB * C, D), jnp.float32),
            pltpu.VMEM((NB * C, D), jnp.float32),
            pltpu.VMEM((NB * TN, D), jnp.float32),
            pltpu.VMEM((NB * TN, D), jnp.float32),
            pltpu.VMEM((NB, NNEG), jnp.float32),
            pltpu.VMEM((NB, NNEG), jnp.float32),
            pltpu.VMEM((LANES,), jnp.float32),
            pltpu.VMEM((LANES,), jnp.float32),
            pltpu.SemaphoreType.DMA,
            pltpu.SemaphoreType.DMA,
            pltpu.SemaphoreType.DMA,
            pltpu.SemaphoreType.DMA,
        ],
    )
    def cbow(ctx_idx_hbm, tn_idx_hbm, tw_hbm, cw_hbm, pos_hbm, neg_hbm,
             idx_c, idx_t, rows_c0, rows_c1, rows_t0, rows_t1,
             neg_buf0, neg_buf1, pos_buf0, pos_buf1,
             semA, semB, semO0, semO1):
        wid = lax.axis_index("s") * NC + lax.axis_index("c")
        wbase = wid * EPW

        pltpu.sync_copy(ctx_idx_hbm.at[pl.ds(wbase * C, EPW * C)], idx_c)
        pltpu.sync_copy(tn_idx_hbm.at[pl.ds(wbase * TN, EPW * TN)], idx_t)

        gbufs = ((rows_c0, rows_t0, semA), (rows_c1, rows_t1, semB))
        obufs = ((neg_buf0, pos_buf0, semO0), (neg_buf1, pos_buf1, semO1))

        def gather_copies(c, buf):
            rows_c, rows_t, sem = buf
            cb = c * (NB * C)
            tb = c * (NB * TN)
            copies = []
            for (o, l) in CTX_SLICES:
                copies.append(pltpu.make_async_copy(
                    cw_hbm.at[idx_c.at[pl.ds(cb + o, l)]],
                    rows_c.at[pl.ds(o, l)], sem))
            for (o, l) in TN_SLICES:
                copies.append(pltpu.make_async_copy(
                    tw_hbm.at[idx_t.at[pl.ds(tb + o, l)]],
                    rows_t.at[pl.ds(o, l)], sem))
            return copies

        def issue(c, buf):
            @pl.when(c < NCH)
            def _():
                for cp in gather_copies(c, buf):
                    cp.start()

        def drain(c, buf):
            for cp in gather_copies(c, buf):
                cp.wait()

        def out_copies(c, obuf):
            neg_buf, pos_buf, sem = obuf
            gbase = wbase + c * NB
            return [
                pltpu.make_async_copy(
                    neg_buf, neg_hbm.at[pl.ds(gbase, NB), :], sem),
                pltpu.make_async_copy(
                    pos_buf.at[pl.ds(0, NB)], pos_hbm.at[pl.ds(gbase, NB)],
                    sem),
            ]

        def start_out(c, obuf):
            for cp in out_copies(c, obuf):
                cp.start()

        def wait_out(c, obuf):
            @pl.when(c >= 0)
            def _():
                for cp in out_copies(c, obuf):
                    cp.wait()

        def compute(c, buf, obuf):
            rows_c, rows_t, _ = buf
            neg_buf, pos_buf, _ = obuf
            lane = lax.iota(jnp.int32, LANES)
            onehot = [lane == jj for jj in range(LANES)]
            rots = [(lane + sh) & (LANES - 1) for sh in (8, 4, 2, 1)]
            zero = jnp.zeros((LANES,), jnp.float32)

            def elem(i, pos_vec):
                ri = i * C
                acc = tuple(_row(rows_c, ri, dd) for dd in range(ND))
                for k in range(1, C):
                    acc = tuple(acc[dd] + _row(rows_c, ri + k, dd)
                                for dd in range(ND))
                scale = jnp.float32(1.0 / C)
                cv = tuple(a * scale for a in acc)

                ti = i * TN
                for g in range(NNEG // LANES):
                    sv = zero
                    for jj in range(LANES):
                        s = _dot_all_lanes(cv, rows_t, ti + 1 + g * LANES + jj,
                                           rots)
                        sv = jnp.where(onehot[jj], s, sv)
                    neg_buf[i, pl.ds(g * LANES, LANES)] = sv
                rem = NNEG % LANES
                rbase = (NNEG // LANES) * LANES
                sv = zero
                for jj in range(rem):
                    s = _dot_all_lanes(cv, rows_t, ti + 1 + rbase + jj, rots)
                    sv = jnp.where(onehot[jj], s, sv)
                plsc.store_scatter(
                    neg_buf,
                    [jnp.full((LANES,), i, jnp.int32), rbase + lane],
                    sv, mask=lane < rem)

                s = _dot_all_lanes(cv, rows_t, ti, rots)
                return jnp.where(lane == i, s, pos_vec)

            pos_vec = lax.fori_loop(0, NB, elem, zero)
            pos_buf[pl.ds(0, LANES)] = pos_vec

        issue(jnp.int32(0), gbufs[0])
        issue(jnp.int32(1), gbufs[1])

        def pair(k, _):
            c0 = 2 * k
            drain(c0, gbufs[0])
            wait_out(c0 - 2, obufs[0])
            compute(c0, gbufs[0], obufs[0])
            start_out(c0, obufs[0])
            issue(c0 + 2, gbufs[0])
            c1 = c0 + 1
            drain(c1, gbufs[1])
            wait_out(c1 - 2, obufs[1])
            compute(c1, gbufs[1], obufs[1])
            start_out(c1, obufs[1])
            issue(c1 + 2, gbufs[1])
            return 0

        lax.fori_loop(0, NCH // 2, pair, 0, unroll=False)
        wait_out(jnp.int32(NCH - 2), obufs[0])
        wait_out(jnp.int32(NCH - 1), obufs[1])

    return cbow


_cbow = _make_kernel()


def kernel(target_indices, context_indices, negative_indices, target_weight,
           context_weight):
    ctx_idx = context_indices.reshape(-1)
    tn_idx = jnp.concatenate([target_indices, negative_indices],
                             axis=1).reshape(-1)
    pos, neg = _cbow(ctx_idx, tn_idx, target_weight, context_weight)
    return pos.reshape(B, 1), neg
